# Initial kernel scaffold; baseline (speedup 1.0000x reference)
import jax
import jax.numpy as jnp
from jax.experimental import pallas as pl


def kernel(teacher_emb, W_down, W_up, gate, token_ids):
    raise NotImplementedError("write your pallas kernel here")


# speedup vs baseline: 3.9307x; 3.9307x over previous
import functools

import jax
import jax.numpy as jnp
from jax import lax
from jax.experimental import pallas as pl
from jax.experimental.pallas import tpu as pltpu
from jax.experimental.pallas import tpu_sc as plsc

_NUM_CORES = 2
_NUM_SUBCORES = 16
_NUM_WORKERS = _NUM_CORES * _NUM_SUBCORES
_CHUNK = 128
_LANES = 128
_HALF = 512


def _down_body(emb_ref, wd_ref, gate_ref, h_ref):
    g = jax.nn.sigmoid(gate_ref[0])
    t = emb_ref[...]
    h_pre = jnp.dot(t, wd_ref[...], preferred_element_type=jnp.float32)
    h = (h_pre * jax.nn.sigmoid(h_pre)) * g
    pad = jnp.zeros((h.shape[0], _LANES - h.shape[1]), jnp.float32)
    h_ref[...] = jnp.concatenate([h, pad], axis=1)


def _up_body(bneck, h_ref, wu_ref, out_ref):
    h = h_ref[...][:, :bneck]
    out_ref[...] = jnp.dot(h, wu_ref[...], preferred_element_type=jnp.float32)


def _make_gather(bneck_pad, n_tokens):
    b_per_w = n_tokens // _NUM_WORKERS
    n_halves = b_per_w // _HALF
    n_chunks = _HALF // _CHUNK
    mesh = plsc.VectorSubcoreMesh(
        core_axis_name="c", subcore_axis_name="s",
        num_cores=_NUM_CORES, num_subcores=_NUM_SUBCORES)

    @functools.partial(
        pl.kernel,
        out_type=jax.ShapeDtypeStruct((n_tokens, bneck_pad), jnp.float32),
        mesh=mesh,
        scratch_types=[
            pltpu.VMEM((b_per_w,), jnp.int32),
            pltpu.VMEM((_HALF, bneck_pad), jnp.float32),
            pltpu.SemaphoreType.DMA,
        ],
    )
    def gather_kernel(table_hbm, idx_hbm, out_hbm, idx_v, rows_v, sem):
        wid = lax.axis_index("s") * _NUM_CORES + lax.axis_index("c")
        base = wid * b_per_w
        pltpu.sync_copy(idx_hbm.at[pl.ds(base, b_per_w)], idx_v)
        for hh in range(n_halves):
            copies = []
            for c in range(n_chunks):
                off = hh * _HALF + c * _CHUNK
                copies.append(pltpu.async_copy(
                    table_hbm.at[idx_v.at[pl.ds(off, _CHUNK)]],
                    rows_v.at[pl.ds(c * _CHUNK, _CHUNK)],
                    sem))
            for cp in copies:
                cp.wait()
            pltpu.sync_copy(rows_v, out_hbm.at[pl.ds(base + hh * _HALF, _HALF)])

    return gather_kernel


def kernel(teacher_emb, W_down, W_up, gate, token_ids):
    vocab, t_dim = teacher_emb.shape
    bneck = W_down.shape[0]
    m_dim = W_up.shape[0]
    b, s = token_ids.shape
    n_tokens = b * s

    wd_t = W_down.T
    wu_t = W_up.T

    vb = 1024
    h_table = pl.pallas_call(
        _down_body,
        grid=(pl.cdiv(vocab, vb),),
        in_specs=[
            pl.BlockSpec((vb, t_dim), lambda i: (i, 0)),
            pl.BlockSpec((t_dim, bneck), lambda i: (0, 0)),
            pl.BlockSpec(memory_space=pltpu.SMEM),
        ],
        out_specs=pl.BlockSpec((vb, _LANES), lambda i: (i, 0)),
        out_shape=jax.ShapeDtypeStruct((vocab, _LANES), jnp.float32),
    )(teacher_emb, wd_t, gate)

    ids_flat = token_ids.reshape(n_tokens)
    h_tok = _make_gather(_LANES, n_tokens)(h_table, ids_flat)

    tb = 1024
    out_flat = pl.pallas_call(
        functools.partial(_up_body, bneck),
        grid=(n_tokens // tb,),
        in_specs=[
            pl.BlockSpec((tb, _LANES), lambda i: (i, 0)),
            pl.BlockSpec((bneck, m_dim), lambda i: (0, 0)),
        ],
        out_specs=pl.BlockSpec((tb, m_dim), lambda i: (i, 0)),
        out_shape=jax.ShapeDtypeStruct((n_tokens, m_dim), jnp.float32),
    )(h_tok, wu_t)

    return out_flat.reshape(b, s, m_dim)


# speedup vs baseline: 4.2865x; 1.0905x over previous
import functools

import jax
import jax.numpy as jnp
from jax import lax
from jax.experimental import pallas as pl
from jax.experimental.pallas import tpu as pltpu
from jax.experimental.pallas import tpu_sc as plsc

_NUM_CORES = 2
_NUM_SUBCORES = 16
_NUM_WORKERS = _NUM_CORES * _NUM_SUBCORES
_CHUNK = 128
_LANES = 128
_HALF = 512


def _down_body(emb_ref, wd_ref, gate_ref, h_ref):
    g = jax.nn.sigmoid(gate_ref[0])
    t = emb_ref[...]
    h_pre = jnp.dot(t, wd_ref[...], preferred_element_type=jnp.float32)
    h = (h_pre * jax.nn.sigmoid(h_pre)) * g
    pad = jnp.zeros((h.shape[0], _LANES - h.shape[1]), jnp.float32)
    h_ref[...] = jnp.concatenate([h, pad], axis=1)


def _up_body(bneck, h_ref, wu_ref, out_ref):
    h = h_ref[...][:, :bneck]
    out_ref[...] = jnp.dot(h, wu_ref[...], preferred_element_type=jnp.float32)


def _make_gather(bneck_pad, n_tokens):
    b_per_w = n_tokens // _NUM_WORKERS
    n_halves = b_per_w // _HALF
    n_chunks = _HALF // _CHUNK
    mesh = plsc.VectorSubcoreMesh(
        core_axis_name="c", subcore_axis_name="s",
        num_cores=_NUM_CORES, num_subcores=_NUM_SUBCORES)

    @functools.partial(
        pl.kernel,
        out_type=jax.ShapeDtypeStruct((n_tokens, bneck_pad), jnp.float32),
        mesh=mesh,
        scratch_types=[
            pltpu.VMEM((b_per_w,), jnp.int32),
            pltpu.VMEM((_HALF, bneck_pad), jnp.float32),
            pltpu.SemaphoreType.DMA,
        ],
    )
    def gather_kernel(table_hbm, idx_hbm, out_hbm, idx_v, rows_v, sem):
        wid = lax.axis_index("s") * _NUM_CORES + lax.axis_index("c")
        base = wid * b_per_w
        pltpu.sync_copy(idx_hbm.at[pl.ds(base, b_per_w)], idx_v)
        for hh in range(n_halves):
            copies = []
            for c in range(n_chunks):
                off = hh * _HALF + c * _CHUNK
                copies.append(pltpu.async_copy(
                    table_hbm.at[idx_v.at[pl.ds(off, _CHUNK)]],
                    rows_v.at[pl.ds(c * _CHUNK, _CHUNK)],
                    sem))
            for cp in copies:
                cp.wait()
            pltpu.sync_copy(rows_v, out_hbm.at[pl.ds(base + hh * _HALF, _HALF)])

    return gather_kernel


def kernel(teacher_emb, W_down, W_up, gate, token_ids):
    vocab, t_dim = teacher_emb.shape
    bneck = W_down.shape[0]
    m_dim = W_up.shape[0]
    b, s = token_ids.shape
    n_tokens = b * s

    wd_t = W_down.T
    wu_t = W_up.T

    vb = 2048
    h_table = pl.pallas_call(
        _down_body,
        grid=(pl.cdiv(vocab, vb),),
        in_specs=[
            pl.BlockSpec((vb, t_dim), lambda i: (i, 0)),
            pl.BlockSpec((t_dim, bneck), lambda i: (0, 0)),
            pl.BlockSpec(memory_space=pltpu.SMEM),
        ],
        out_specs=pl.BlockSpec((vb, _LANES), lambda i: (i, 0)),
        out_shape=jax.ShapeDtypeStruct((vocab, _LANES), jnp.float32),
    )(teacher_emb, wd_t, gate)

    ids_flat = token_ids.reshape(n_tokens)
    h_tok = _make_gather(_LANES, n_tokens)(h_table, ids_flat)

    tb = 2048
    out_flat = pl.pallas_call(
        functools.partial(_up_body, bneck),
        grid=(n_tokens // tb,),
        in_specs=[
            pl.BlockSpec((tb, _LANES), lambda i: (i, 0)),
            pl.BlockSpec((bneck, m_dim), lambda i: (0, 0)),
        ],
        out_specs=pl.BlockSpec((tb, m_dim), lambda i: (i, 0)),
        out_shape=jax.ShapeDtypeStruct((n_tokens, m_dim), jnp.float32),
    )(h_tok, wu_t)

    return out_flat.reshape(b, s, m_dim)
